# baseline (speedup 1.0000x reference)
import functools

import jax
import jax.numpy as jnp
from jax import lax
from jax.experimental import pallas as pl
from jax.experimental.pallas import tpu as pltpu
from jax.experimental.pallas import tpu_sc as plsc

N0, N1, N2 = 50000, 10000, 4096
E1, E2 = 320000, 131072
D = 128
DP = 144
NC, NS = 2, 16


def _make_sc_agg(E, NP, C, NB, P=1, interpret=False):
    EW = E // (NC * NS)
    n_chunks = EW // C
    assert n_chunks * C == EW
    n_cph = n_chunks // P
    assert n_cph * P == n_chunks
    n_groups = n_cph // NB
    assert n_groups * NB == n_cph and n_groups % 2 == 0
    n_pairs = n_groups // 2
    RPT = NP // NS
    assert RPT * NS == NP and RPT % 8 == 0
    mesh = plsc.VectorSubcoreMesh(core_axis_name="c", subcore_axis_name="s",
                                  num_cores=NC, num_subcores=NS)

    @functools.partial(
        pl.kernel,
        out_type=jax.ShapeDtypeStruct((NC, NP, DP), jnp.float32),
        mesh=mesh,
        scratch_types=[
            pltpu.VMEM((n_cph, C), jnp.int32),
            pltpu.VMEM((n_cph, C), jnp.int32),
            pltpu.VMEM((2, NB, C, DP), jnp.float32),
            pltpu.VMEM_SHARED((NP, DP), jnp.float32),
            pltpu.SemaphoreType.DMA,
            pltpu.SemaphoreType.DMA,
            pltpu.SemaphoreType.DMA,
            pltpu.SemaphoreType.DMA,
        ],
        compiler_params=pltpu.CompilerParams(use_tc_tiling_on_sc=False),
        interpret=interpret,
    )
    def agg_kernel(table, srcR, dstR, zeros, out,
                   idxs_v, idxd_v, bufs, acc_sh, g0, g1, s0, s1):
        cid = lax.axis_index("c")
        sid = lax.axis_index("s")
        w = cid * NS + sid
        gsem = (g0, g1)
        ssem = (s0, s1)
        pltpu.sync_copy(zeros.at[pl.ds(sid * RPT, RPT)],
                        acc_sh.at[pl.ds(sid * RPT, RPT)])
        plsc.subcore_barrier()

        def gather(c, p, b):
            return pltpu.async_copy(table.at[idxs_v.at[c]],
                                    bufs.at[p].at[b], gsem[p])

        def scatter(c, p, b):
            return pltpu.async_copy(bufs.at[p].at[b],
                                    acc_sh.at[idxd_v.at[c]], ssem[p],
                                    add=True)

        for ph in range(P):
            row0 = w * n_chunks + ph * n_cph
            pltpu.sync_copy(srcR.at[pl.ds(row0, n_cph)], idxs_v)
            pltpu.sync_copy(dstR.at[pl.ds(row0, n_cph)], idxd_v)

            for p in (0, 1):
                for b in range(NB):
                    gather(p * NB + b, p, b)

            def pair_body(q, carry):
                for p in (0, 1):
                    base_c = (2 * q + p) * NB
                    for b in range(NB):
                        c = base_c + b
                        pltpu.make_async_copy(table.at[idxs_v.at[c]],
                                              bufs.at[p].at[b],
                                              gsem[p]).wait()
                        scatter(c, p, b)
                    for b in range(NB):
                        c = base_c + b
                        pltpu.make_async_copy(bufs.at[p].at[b],
                                              acc_sh.at[idxd_v.at[c]],
                                              ssem[p]).wait()

                        @pl.when(q < n_pairs - 1)
                        def _():
                            gather(c + 2 * NB, p, b)
                return carry

            lax.fori_loop(0, n_pairs, pair_body, 0)
        plsc.subcore_barrier()
        pltpu.sync_copy(acc_sh.at[pl.ds(sid * RPT, RPT)],
                        out.at[cid, pl.ds(sid * RPT, RPT)])

    return agg_kernel


def _dense(parts, xdst, wlT, wrT, b, relu, pad_out, BR, interpret=False):
    N = xdst.shape[0]
    assert N % BR == 0
    DO = DP if pad_out else D

    def body(p_ref, xd_ref, wl_ref, wr_ref, b_ref, o_ref):
        agg = p_ref[0] + p_ref[1]
        cnt = jnp.maximum(agg[:, D:D + 1], 1.0)
        mean = agg[:, :D] / cnt
        h = jnp.dot(mean, wl_ref[...], preferred_element_type=jnp.float32)
        h = h + jnp.dot(xd_ref[...], wr_ref[...],
                        preferred_element_type=jnp.float32)
        h = h + b_ref[...]
        if relu:
            h = jnp.maximum(h, 0.0)
        if pad_out:
            col = lax.broadcasted_iota(jnp.int32, (BR, DP - D), 1) == 0
            h = jnp.concatenate([h, col.astype(jnp.float32)], axis=1)
        o_ref[...] = h

    return pl.pallas_call(
        body,
        grid=(N // BR,),
        in_specs=[
            pl.BlockSpec((NC, BR, DP), lambda i: (0, i, 0)),
            pl.BlockSpec((BR, D), lambda i: (i, 0)),
            pl.BlockSpec((D, D), lambda i: (0, 0)),
            pl.BlockSpec((D, D), lambda i: (0, 0)),
            pl.BlockSpec((1, D), lambda i: (0, 0)),
        ],
        out_specs=pl.BlockSpec((BR, DO), lambda i: (i, 0)),
        out_shape=jax.ShapeDtypeStruct((N, DO), jnp.float32),
        interpret=interpret,
    )(parts, xdst, wlT, wrT, b)


def kernel(x, edge_index1, edge_index2, W_l1, b_l1, W_r1, W_l2, b_l2, W_r2):
    src1 = edge_index1[0].astype(jnp.int32)
    dst1 = edge_index1[1].astype(jnp.int32)
    src2 = edge_index2[0].astype(jnp.int32)
    dst2 = edge_index2[1].astype(jnp.int32)

    onescol = (jnp.arange(DP - D)[None, :] == 0).astype(jnp.float32)
    xe = jnp.concatenate([x, jnp.broadcast_to(onescol, (N0, DP - D))], axis=1)
    N1P = 10112
    z1 = jnp.zeros((N1P, DP), jnp.float32)
    z2 = jnp.zeros((N2, DP), jnp.float32)

    E1P = 327680
    src1p = jnp.concatenate([src1, jnp.zeros((E1P - E1,), jnp.int32)])
    dpad = N1 + jnp.arange(E1P - E1, dtype=jnp.int32) % (N1P - N1)
    dst1p = jnp.concatenate([dst1, dpad])

    parts1 = _make_sc_agg(E1P, N1P, 40, 2, P=2)(
        xe, src1p.reshape(-1, 40), dst1p.reshape(-1, 40), z1)
    he = _dense(parts1, x[:N1], W_l1.T, W_r1.T, b_l1[None, :],
                relu=True, pad_out=True, BR=1000)
    parts2 = _make_sc_agg(E2, N2, 64, 4)(
        he, src2.reshape(-1, 64), dst2.reshape(-1, 64), z2)

    h2 = _dense(parts2, he[:N2, :D], W_l2.T, W_r2.T, b_l2[None, :],
                relu=False, pad_out=False, BR=1024)
    out = he[:, :D]
    return (h2, h2, out)


# speedup vs baseline: 1.0198x; 1.0198x over previous
import functools

import jax
import jax.numpy as jnp
from jax import lax
from jax.experimental import pallas as pl
from jax.experimental.pallas import tpu as pltpu
from jax.experimental.pallas import tpu_sc as plsc

N0, N1, N2 = 50000, 10000, 4096
E1, E2 = 320000, 131072
D = 128
DP = 144
NC, NS = 2, 16


def _make_sc_agg(E, NP, C, NB, P=1, interpret=False):
    EW = E // (NC * NS)
    n_chunks = EW // C
    assert n_chunks * C == EW
    n_cph = n_chunks // P
    assert n_cph * P == n_chunks
    n_groups = n_cph // NB
    assert n_groups * NB == n_cph and n_groups % 2 == 0
    n_pairs = n_groups // 2
    RPT = NP // NS
    assert RPT * NS == NP and RPT % 8 == 0
    mesh = plsc.VectorSubcoreMesh(core_axis_name="c", subcore_axis_name="s",
                                  num_cores=NC, num_subcores=NS)

    @functools.partial(
        pl.kernel,
        out_type=jax.ShapeDtypeStruct((NC, NP, DP), jnp.float32),
        mesh=mesh,
        scratch_types=[
            pltpu.VMEM((n_cph, C), jnp.int32),
            pltpu.VMEM((n_cph, C), jnp.int32),
            pltpu.VMEM((2, NB, C, DP), jnp.float32),
            pltpu.VMEM_SHARED((NP, DP), jnp.float32),
            pltpu.SemaphoreType.DMA,
            pltpu.SemaphoreType.DMA,
            pltpu.SemaphoreType.DMA,
            pltpu.SemaphoreType.DMA,
        ],
        compiler_params=pltpu.CompilerParams(use_tc_tiling_on_sc=False),
        interpret=interpret,
    )
    def agg_kernel(table, srcR, dstR, zeros, out,
                   idxs_v, idxd_v, bufs, acc_sh, g0, g1, s0, s1):
        cid = lax.axis_index("c")
        sid = lax.axis_index("s")
        w = cid * NS + sid
        gsem = (g0, g1)
        ssem = (s0, s1)
        pltpu.sync_copy(zeros.at[pl.ds(sid * RPT, RPT)],
                        acc_sh.at[pl.ds(sid * RPT, RPT)])
        plsc.subcore_barrier()

        def gather(c, p, b):
            return pltpu.async_copy(table.at[idxs_v.at[c]],
                                    bufs.at[p].at[b], gsem[p])

        def scatter(c, p, b):
            return pltpu.async_copy(bufs.at[p].at[b],
                                    acc_sh.at[idxd_v.at[c]], ssem[p],
                                    add=True)

        for ph in range(P):
            row0 = w * n_chunks + ph * n_cph
            pltpu.sync_copy(srcR.at[pl.ds(row0, n_cph)], idxs_v)
            pltpu.sync_copy(dstR.at[pl.ds(row0, n_cph)], idxd_v)

            for p in (0, 1):
                for b in range(NB):
                    gather(p * NB + b, p, b)

            def pair_body(q, carry):
                for p in (0, 1):
                    base_c = (2 * q + p) * NB
                    for b in range(NB):
                        c = base_c + b
                        pltpu.make_async_copy(table.at[idxs_v.at[c]],
                                              bufs.at[p].at[b],
                                              gsem[p]).wait()
                        scatter(c, p, b)
                    for b in range(NB):
                        c = base_c + b
                        pltpu.make_async_copy(bufs.at[p].at[b],
                                              acc_sh.at[idxd_v.at[c]],
                                              ssem[p]).wait()

                        @pl.when(q < n_pairs - 1)
                        def _():
                            gather(c + 2 * NB, p, b)
                return carry

            lax.fori_loop(0, n_pairs, pair_body, 0)
        plsc.subcore_barrier()
        pltpu.sync_copy(acc_sh.at[pl.ds(sid * RPT, RPT)],
                        out.at[cid, pl.ds(sid * RPT, RPT)])

    return agg_kernel


def _dense(parts, xdst, wlT, wrT, b, relu, pad_out, BR, interpret=False):
    N = xdst.shape[0]
    assert N % BR == 0
    DO = DP if pad_out else D

    def body(p_ref, xd_ref, wl_ref, wr_ref, b_ref, o_ref):
        agg = p_ref[0] + p_ref[1]
        cnt = jnp.maximum(agg[:, D:D + 1], 1.0)
        mean = agg[:, :D] / cnt
        h = jnp.dot(mean, wl_ref[...], preferred_element_type=jnp.float32)
        h = h + jnp.dot(xd_ref[...], wr_ref[...],
                        preferred_element_type=jnp.float32)
        h = h + b_ref[...]
        if relu:
            h = jnp.maximum(h, 0.0)
        if pad_out:
            col = lax.broadcasted_iota(jnp.int32, (BR, DP - D), 1) == 0
            h = jnp.concatenate([h, col.astype(jnp.float32)], axis=1)
        o_ref[...] = h

    return pl.pallas_call(
        body,
        grid=(N // BR,),
        in_specs=[
            pl.BlockSpec((NC, BR, DP), lambda i: (0, i, 0)),
            pl.BlockSpec((BR, D), lambda i: (i, 0)),
            pl.BlockSpec((D, D), lambda i: (0, 0)),
            pl.BlockSpec((D, D), lambda i: (0, 0)),
            pl.BlockSpec((1, D), lambda i: (0, 0)),
        ],
        out_specs=pl.BlockSpec((BR, DO), lambda i: (i, 0)),
        out_shape=jax.ShapeDtypeStruct((N, DO), jnp.float32),
        interpret=interpret,
    )(parts, xdst, wlT, wrT, b)


def kernel(x, edge_index1, edge_index2, W_l1, b_l1, W_r1, W_l2, b_l2, W_r2):
    src1 = edge_index1[0].astype(jnp.int32)
    dst1 = edge_index1[1].astype(jnp.int32)
    src2 = edge_index2[0].astype(jnp.int32)
    dst2 = edge_index2[1].astype(jnp.int32)

    onescol = (jnp.arange(DP - D)[None, :] == 0).astype(jnp.float32)
    xe = jnp.concatenate([x, jnp.broadcast_to(onescol, (N0, DP - D))], axis=1)
    xe = jnp.concatenate([xe, jnp.zeros((8, DP), jnp.float32)], axis=0)
    N1P = 10112
    z1 = jnp.zeros((N1P, DP), jnp.float32)
    z2 = jnp.zeros((N2, DP), jnp.float32)

    E1P = 327680
    src1p = jnp.concatenate([src1, jnp.full((E1P - E1,), N0, jnp.int32)])
    dpad = jnp.arange(E1P - E1, dtype=jnp.int32) * 97 % N1P
    dst1p = jnp.concatenate([dst1, dpad])

    parts1 = _make_sc_agg(E1P, N1P, 40, 2, P=2)(
        xe, src1p.reshape(-1, 40), dst1p.reshape(-1, 40), z1)
    he = _dense(parts1, x[:N1], W_l1.T, W_r1.T, b_l1[None, :],
                relu=True, pad_out=True, BR=1000)
    parts2 = _make_sc_agg(E2, N2, 64, 4)(
        he, src2.reshape(-1, 64), dst2.reshape(-1, 64), z2)

    h2 = _dense(parts2, he[:N2, :D], W_l2.T, W_r2.T, b_l2[None, :],
                relu=False, pad_out=False, BR=1024)
    out = he[:, :D]
    return (h2, h2, out)


# speedup vs baseline: 1.7336x; 1.6999x over previous
import functools

import jax
import jax.numpy as jnp
from jax import lax
from jax.experimental import pallas as pl
from jax.experimental.pallas import tpu as pltpu
from jax.experimental.pallas import tpu_sc as plsc

N0, N1, N2 = 50000, 10000, 4096
E1, E2 = 320000, 131072
D = 128
DP = 144
NC, NS = 2, 16


def _make_sc_agg(E, NP, C, NB, P=1, interpret=False):
    EW = E // (NC * NS)
    n_chunks = EW // C
    assert n_chunks * C == EW
    n_cph = n_chunks // P
    assert n_cph * P == n_chunks
    n_groups = n_cph // NB
    assert n_groups * NB == n_cph and n_groups % 2 == 0
    n_pairs = n_groups // 2
    RPT = NP // NS
    assert RPT * NS == NP and RPT % 8 == 0
    mesh = plsc.VectorSubcoreMesh(core_axis_name="c", subcore_axis_name="s",
                                  num_cores=NC, num_subcores=NS)

    @functools.partial(
        pl.kernel,
        out_type=jax.ShapeDtypeStruct((NC, NP, DP), jnp.float32),
        mesh=mesh,
        scratch_types=[
            pltpu.VMEM((n_cph, C), jnp.int32),
            pltpu.VMEM((n_cph, C), jnp.int32),
            pltpu.VMEM((2, NB, C, DP), jnp.float32),
            pltpu.VMEM_SHARED((NP, DP), jnp.float32),
            pltpu.SemaphoreType.DMA,
            pltpu.SemaphoreType.DMA,
            pltpu.SemaphoreType.DMA,
            pltpu.SemaphoreType.DMA,
        ],
        compiler_params=pltpu.CompilerParams(use_tc_tiling_on_sc=False),
        interpret=interpret,
    )
    def agg_kernel(table, srcR, dstR, zeros, out,
                   idxs_v, idxd_v, bufs, acc_sh, g0, g1, s0, s1):
        cid = lax.axis_index("c")
        sid = lax.axis_index("s")
        w = cid * NS + sid
        gsem = (g0, g1)
        ssem = (s0, s1)
        pltpu.sync_copy(zeros.at[pl.ds(sid * RPT, RPT)],
                        acc_sh.at[pl.ds(sid * RPT, RPT)])
        plsc.subcore_barrier()

        def gather(c, p, b):
            return pltpu.async_copy(table.at[idxs_v.at[c]],
                                    bufs.at[p].at[b], gsem[p])

        def scatter(c, p, b):
            return pltpu.async_copy(bufs.at[p].at[b],
                                    acc_sh.at[idxd_v.at[c]], ssem[p],
                                    add=True)

        for ph in range(P):
            row0 = w * n_chunks + ph * n_cph
            pltpu.sync_copy(srcR.at[pl.ds(row0, n_cph)], idxs_v)
            pltpu.sync_copy(dstR.at[pl.ds(row0, n_cph)], idxd_v)

            for p in (0, 1):
                for b in range(NB):
                    gather(p * NB + b, p, b)

            def pair_body(q, carry):
                for p in (0, 1):
                    base_c = (2 * q + p) * NB
                    for b in range(NB):
                        c = base_c + b
                        pltpu.make_async_copy(table.at[idxs_v.at[c]],
                                              bufs.at[p].at[b],
                                              gsem[p]).wait()
                        scatter(c, p, b)
                    for b in range(NB):
                        c = base_c + b
                        pltpu.make_async_copy(bufs.at[p].at[b],
                                              acc_sh.at[idxd_v.at[c]],
                                              ssem[p]).wait()

                        @pl.when(q < n_pairs - 1)
                        def _():
                            gather(c + 2 * NB, p, b)
                return carry

            lax.fori_loop(0, n_pairs, pair_body, 0)
        plsc.subcore_barrier()
        pltpu.sync_copy(acc_sh.at[pl.ds(sid * RPT, RPT)],
                        out.at[cid, pl.ds(sid * RPT, RPT)])

    return agg_kernel


def _dense(parts, xdst, wlT, wrT, b, relu, pad_out, BR, interpret=False):
    N = xdst.shape[0]
    assert N % BR == 0
    DO = DP if pad_out else D

    def body(p_ref, xd_ref, wl_ref, wr_ref, b_ref, o_ref):
        agg = p_ref[0] + p_ref[1]
        cnt = jnp.maximum(agg[:, D:D + 1], 1.0)
        mean = agg[:, :D] / cnt
        h = jnp.dot(mean, wl_ref[...], preferred_element_type=jnp.float32)
        h = h + jnp.dot(xd_ref[...], wr_ref[...],
                        preferred_element_type=jnp.float32)
        h = h + b_ref[...]
        if relu:
            h = jnp.maximum(h, 0.0)
        if pad_out:
            col = lax.broadcasted_iota(jnp.int32, (BR, DP - D), 1) == 0
            h = jnp.concatenate([h, col.astype(jnp.float32)], axis=1)
        o_ref[...] = h

    return pl.pallas_call(
        body,
        grid=(N // BR,),
        in_specs=[
            pl.BlockSpec((NC, BR, DP), lambda i: (0, i, 0)),
            pl.BlockSpec((BR, D), lambda i: (i, 0)),
            pl.BlockSpec((D, D), lambda i: (0, 0)),
            pl.BlockSpec((D, D), lambda i: (0, 0)),
            pl.BlockSpec((1, D), lambda i: (0, 0)),
        ],
        out_specs=pl.BlockSpec((BR, DO), lambda i: (i, 0)),
        out_shape=jax.ShapeDtypeStruct((N, DO), jnp.float32),
        interpret=interpret,
    )(parts, xdst, wlT, wrT, b)


def kernel(x, edge_index1, edge_index2, W_l1, b_l1, W_r1, W_l2, b_l2, W_r2):
    src1 = edge_index1[0].astype(jnp.int32)
    dst1 = edge_index1[1].astype(jnp.int32)
    src2 = edge_index2[0].astype(jnp.int32)
    dst2 = edge_index2[1].astype(jnp.int32)

    onescol = (jnp.arange(DP - D)[None, :] == 0).astype(jnp.float32)
    xe = jnp.concatenate([x, jnp.broadcast_to(onescol, (N0, DP - D))], axis=1)
    N1P = 10112
    z1 = jnp.zeros((N1P, DP), jnp.float32)
    z2 = jnp.zeros((N2, DP), jnp.float32)

    parts1 = _make_sc_agg(E1, N1P, 40, 1)(
        xe, src1.reshape(-1, 40), dst1.reshape(-1, 40), z1)
    he = _dense(parts1, x[:N1], W_l1.T, W_r1.T, b_l1[None, :],
                relu=True, pad_out=True, BR=1000)
    parts2 = _make_sc_agg(E2, N2, 64, 4)(
        he, src2.reshape(-1, 64), dst2.reshape(-1, 64), z2)

    h2 = _dense(parts2, he[:N2, :D], W_l2.T, W_r2.T, b_l2[None, :],
                relu=False, pad_out=False, BR=1024)
    out = he[:, :D]
    return (h2, h2, out)


# speedup vs baseline: 2.0538x; 1.1847x over previous
import functools

import jax
import jax.numpy as jnp
from jax import lax
from jax.experimental import pallas as pl
from jax.experimental.pallas import tpu as pltpu
from jax.experimental.pallas import tpu_sc as plsc

N0, N1, N2 = 50000, 10000, 4096
E1, E2 = 320000, 131072
D = 128
DP = 144
NC, NS = 2, 16


def _make_sc_agg(E, NP, C, S, P=1, interpret=False):
    EW = E // (NC * NS)
    n_chunks = EW // C
    assert n_chunks * C == EW
    n_cph = n_chunks // P
    assert n_cph * P == n_chunks
    n_rounds = n_cph // S
    assert n_rounds * S == n_cph and n_rounds >= 2
    RPT = NP // NS
    assert RPT * NS == NP and RPT % 8 == 0
    mesh = plsc.VectorSubcoreMesh(core_axis_name="c", subcore_axis_name="s",
                                  num_cores=NC, num_subcores=NS)

    @functools.partial(
        pl.kernel,
        out_type=jax.ShapeDtypeStruct((NC, NP, DP), jnp.float32),
        mesh=mesh,
        scratch_types=(
            [pltpu.VMEM((n_cph, C), jnp.int32),
             pltpu.VMEM((n_cph, C), jnp.int32),
             pltpu.VMEM((S, C, DP), jnp.float32),
             pltpu.VMEM_SHARED((NP, DP), jnp.float32)]
            + [pltpu.SemaphoreType.DMA] * (2 * S)
        ),
        compiler_params=pltpu.CompilerParams(use_tc_tiling_on_sc=False),
        interpret=interpret,
    )
    def agg_kernel(table, srcR, dstR, zeros, out,
                   idxs_v, idxd_v, bufs, acc_sh, *sems):
        cid = lax.axis_index("c")
        sid = lax.axis_index("s")
        w = cid * NS + sid
        gsem = sems[:S]
        ssem = sems[S:]
        pltpu.sync_copy(zeros.at[pl.ds(sid * RPT, RPT)],
                        acc_sh.at[pl.ds(sid * RPT, RPT)])
        plsc.subcore_barrier()

        def gather(c, s):
            return pltpu.async_copy(table.at[idxs_v.at[c]],
                                    bufs.at[s], gsem[s])

        def scatter(c, s):
            return pltpu.async_copy(bufs.at[s],
                                    acc_sh.at[idxd_v.at[c]], ssem[s],
                                    add=True)

        for ph in range(P):
            row0 = w * n_chunks + ph * n_cph
            pltpu.sync_copy(srcR.at[pl.ds(row0, n_cph)], idxs_v)
            pltpu.sync_copy(dstR.at[pl.ds(row0, n_cph)], idxd_v)

            for s in range(S):
                gather(s, s)

            def round_body(r, carry):
                for s in range(S):
                    c = r * S + s
                    pltpu.make_async_copy(table.at[idxs_v.at[c]],
                                          bufs.at[s], gsem[s]).wait()
                    scatter(c, s)
                    pltpu.make_async_copy(bufs.at[s],
                                          acc_sh.at[idxd_v.at[c]],
                                          ssem[s]).wait()

                    @pl.when(r < n_rounds - 1)
                    def _():
                        gather(c + S, s)
                return carry

            lax.fori_loop(0, n_rounds, round_body, 0)
        plsc.subcore_barrier()
        pltpu.sync_copy(acc_sh.at[pl.ds(sid * RPT, RPT)],
                        out.at[cid, pl.ds(sid * RPT, RPT)])

    return agg_kernel


def _dense(parts, xdst, wlT, wrT, b, relu, pad_out, BR, interpret=False):
    N = xdst.shape[0]
    assert N % BR == 0
    DO = DP if pad_out else D

    def body(p_ref, xd_ref, wl_ref, wr_ref, b_ref, o_ref):
        agg = p_ref[0] + p_ref[1]
        cnt = jnp.maximum(agg[:, D:D + 1], 1.0)
        mean = agg[:, :D] / cnt
        h = jnp.dot(mean, wl_ref[...], preferred_element_type=jnp.float32)
        h = h + jnp.dot(xd_ref[...], wr_ref[...],
                        preferred_element_type=jnp.float32)
        h = h + b_ref[...]
        if relu:
            h = jnp.maximum(h, 0.0)
        if pad_out:
            col = lax.broadcasted_iota(jnp.int32, (BR, DP - D), 1) == 0
            h = jnp.concatenate([h, col.astype(jnp.float32)], axis=1)
        o_ref[...] = h

    return pl.pallas_call(
        body,
        grid=(N // BR,),
        in_specs=[
            pl.BlockSpec((NC, BR, DP), lambda i: (0, i, 0)),
            pl.BlockSpec((BR, D), lambda i: (i, 0)),
            pl.BlockSpec((D, D), lambda i: (0, 0)),
            pl.BlockSpec((D, D), lambda i: (0, 0)),
            pl.BlockSpec((1, D), lambda i: (0, 0)),
        ],
        out_specs=pl.BlockSpec((BR, DO), lambda i: (i, 0)),
        out_shape=jax.ShapeDtypeStruct((N, DO), jnp.float32),
        interpret=interpret,
    )(parts, xdst, wlT, wrT, b)


def kernel(x, edge_index1, edge_index2, W_l1, b_l1, W_r1, W_l2, b_l2, W_r2):
    src1 = edge_index1[0].astype(jnp.int32)
    dst1 = edge_index1[1].astype(jnp.int32)
    src2 = edge_index2[0].astype(jnp.int32)
    dst2 = edge_index2[1].astype(jnp.int32)

    onescol = (jnp.arange(DP - D)[None, :] == 0).astype(jnp.float32)
    xe = jnp.concatenate([x, jnp.broadcast_to(onescol, (N0, DP - D))], axis=1)
    N1P = 10112
    z1 = jnp.zeros((N1P, DP), jnp.float32)
    z2 = jnp.zeros((N2, DP), jnp.float32)

    parts1 = _make_sc_agg(E1, N1P, 40, 5, P=2)(
        xe, src1.reshape(-1, 40), dst1.reshape(-1, 40), z1)
    he = _dense(parts1, x[:N1], W_l1.T, W_r1.T, b_l1[None, :],
                relu=True, pad_out=True, BR=1000)
    parts2 = _make_sc_agg(E2, N2, 64, 8)(
        he, src2.reshape(-1, 64), dst2.reshape(-1, 64), z2)

    h2 = _dense(parts2, he[:N2, :D], W_l2.T, W_r2.T, b_l2[None, :],
                relu=False, pad_out=False, BR=1024)
    out = he[:, :D]
    return (h2, h2, out)


# speedup vs baseline: 2.5197x; 1.2268x over previous
import functools

import jax
import jax.numpy as jnp
from jax import lax
from jax.experimental import pallas as pl
from jax.experimental.pallas import tpu as pltpu
from jax.experimental.pallas import tpu_sc as plsc

N0, N1, N2 = 50000, 10000, 4096
E1, E2 = 320000, 131072
D = 128
CW = 16
NC, NS = 2, 16


def _make_sc_agg(E, NP, C, S):
    EW = E // (NC * NS)
    n_chunks = EW // C
    assert n_chunks * C == EW
    n_rounds = n_chunks // S
    assert n_rounds * S == n_chunks and n_rounds >= 2
    RPT = NP // NS
    assert RPT * NS == NP and RPT % 8 == 0
    mesh = plsc.VectorSubcoreMesh(core_axis_name="c", subcore_axis_name="s",
                                  num_cores=NC, num_subcores=NS)

    @functools.partial(
        pl.kernel,
        out_type=jax.ShapeDtypeStruct((NC, NP, D), jnp.float32),
        mesh=mesh,
        scratch_types=(
            [pltpu.VMEM((S, C, D), jnp.float32)]
            + [pltpu.VMEM((C,), jnp.int32)] * S
            + [pltpu.VMEM((C,), jnp.int32)] * S
            + [pltpu.VMEM_SHARED((NP, D), jnp.float32)]
            + [pltpu.SemaphoreType.DMA] * (3 * S)
        ),
    )
    def agg_kernel(table, src, dst, zeros, out, bufs, *rest):
        idxs = rest[:S]
        idxd = rest[S:2 * S]
        acc_sh = rest[2 * S]
        isem = rest[2 * S + 1:2 * S + 1 + S]
        gsem = rest[2 * S + 1 + S:2 * S + 1 + 2 * S]
        ssem = rest[2 * S + 1 + 2 * S:]
        cid = lax.axis_index("c")
        sid = lax.axis_index("s")
        base = (cid * NS + sid) * EW
        pltpu.sync_copy(zeros.at[pl.ds(sid * RPT, RPT)],
                        acc_sh.at[pl.ds(sid * RPT, RPT)])
        plsc.subcore_barrier()

        def round_body(r, carry):
            for s in range(S):
                c = r * S + s

                @pl.when(r > 0)
                def _():
                    pltpu.make_async_copy(
                        bufs.at[s], acc_sh.at[idxd[s]], ssem[s]).wait()
                pltpu.async_copy(src.at[pl.ds(base + c * C, C)],
                                 idxs[s], isem[s])
                pltpu.async_copy(dst.at[pl.ds(base + c * C, C)],
                                 idxd[s], isem[s])
            for s in range(S):
                c = r * S + s
                pltpu.make_async_copy(src.at[pl.ds(base + c * C, C)],
                                      idxs[s], isem[s]).wait()
                pltpu.make_async_copy(dst.at[pl.ds(base + c * C, C)],
                                      idxd[s], isem[s]).wait()
                pltpu.async_copy(table.at[idxs[s]], bufs.at[s], gsem[s])
            for s in range(S):
                pltpu.make_async_copy(table.at[idxs[s]], bufs.at[s],
                                      gsem[s]).wait()
                pltpu.async_copy(bufs.at[s], acc_sh.at[idxd[s]], ssem[s],
                                 add=True)
            return carry

        lax.fori_loop(0, n_rounds, round_body, 0)
        for s in range(S):
            pltpu.make_async_copy(bufs.at[s], acc_sh.at[idxd[s]],
                                  ssem[s]).wait()
        plsc.subcore_barrier()
        pltpu.sync_copy(acc_sh.at[pl.ds(sid * RPT, RPT)],
                        out.at[cid, pl.ds(sid * RPT, RPT)])

    return agg_kernel


def _make_sc_cnt(E, NP, C, S):
    EW = E // (NC * NS)
    n_chunks = EW // C
    assert n_chunks * C == EW
    RPT = NP // NS
    assert RPT * NS == NP and RPT % 8 == 0
    mesh = plsc.VectorSubcoreMesh(core_axis_name="c", subcore_axis_name="s",
                                  num_cores=NC, num_subcores=NS)

    @functools.partial(
        pl.kernel,
        out_type=jax.ShapeDtypeStruct((NC, NP, CW), jnp.float32),
        mesh=mesh,
        scratch_types=[
            pltpu.VMEM((n_chunks, C), jnp.int32),
            pltpu.VMEM((C, CW), jnp.float32),
            pltpu.VMEM_SHARED((NP, CW), jnp.float32),
            pltpu.SemaphoreType.DMA,
        ],
        compiler_params=pltpu.CompilerParams(use_tc_tiling_on_sc=False),
    )
    def cnt_kernel(dstR, ones, zeros, out, idxd_v, ones_v, acc_sh, ssem):
        cid = lax.axis_index("c")
        sid = lax.axis_index("s")
        w = cid * NS + sid
        pltpu.sync_copy(zeros.at[pl.ds(sid * RPT, RPT)],
                        acc_sh.at[pl.ds(sid * RPT, RPT)])
        pltpu.sync_copy(dstR.at[pl.ds(w * n_chunks, n_chunks)], idxd_v)
        pltpu.sync_copy(ones, ones_v)
        plsc.subcore_barrier()

        def scatter(c):
            return pltpu.async_copy(ones_v, acc_sh.at[idxd_v.at[c]], ssem,
                                    add=True)

        for c in range(S):
            scatter(c)

        def body(c, carry):
            pltpu.make_async_copy(ones_v, acc_sh.at[idxd_v.at[c]],
                                  ssem).wait()
            scatter(c + S)
            return carry

        lax.fori_loop(0, n_chunks - S, body, 0)
        for c in range(S):
            pltpu.make_async_copy(ones_v, acc_sh.at[idxd_v.at[0]],
                                  ssem).wait()
        plsc.subcore_barrier()
        pltpu.sync_copy(acc_sh.at[pl.ds(sid * RPT, RPT)],
                        out.at[cid, pl.ds(sid * RPT, RPT)])

    return cnt_kernel


def _dense(parts, cnts, xdst, wlT, wrT, b, relu, BR):
    N = xdst.shape[0]
    assert N % BR == 0

    def body(p_ref, c_ref, xd_ref, wl_ref, wr_ref, b_ref, o_ref):
        agg = p_ref[0] + p_ref[1]
        cnt = jnp.maximum(c_ref[0, :, 0:1] + c_ref[1, :, 0:1], 1.0)
        mean = agg / cnt
        h = jnp.dot(mean, wl_ref[...], preferred_element_type=jnp.float32)
        h = h + jnp.dot(xd_ref[...], wr_ref[...],
                        preferred_element_type=jnp.float32)
        h = h + b_ref[...]
        if relu:
            h = jnp.maximum(h, 0.0)
        o_ref[...] = h

    return pl.pallas_call(
        body,
        grid=(N // BR,),
        in_specs=[
            pl.BlockSpec((NC, BR, D), lambda i: (0, i, 0)),
            pl.BlockSpec((NC, BR, CW), lambda i: (0, i, 0)),
            pl.BlockSpec((BR, D), lambda i: (i, 0)),
            pl.BlockSpec((D, D), lambda i: (0, 0)),
            pl.BlockSpec((D, D), lambda i: (0, 0)),
            pl.BlockSpec((1, D), lambda i: (0, 0)),
        ],
        out_specs=pl.BlockSpec((BR, D), lambda i: (i, 0)),
        out_shape=jax.ShapeDtypeStruct((N, D), jnp.float32),
    )(parts, cnts, xdst, wlT, wrT, b)


def kernel(x, edge_index1, edge_index2, W_l1, b_l1, W_r1, W_l2, b_l2, W_r2):
    src1 = edge_index1[0].astype(jnp.int32)
    dst1 = edge_index1[1].astype(jnp.int32)
    src2 = edge_index2[0].astype(jnp.int32)
    dst2 = edge_index2[1].astype(jnp.int32)

    N1P = 10112
    z1 = jnp.zeros((N1P, D), jnp.float32)
    z2 = jnp.zeros((N2, D), jnp.float32)
    zc1 = jnp.zeros((N1P, CW), jnp.float32)
    zc2 = jnp.zeros((N2, CW), jnp.float32)
    ones1 = jnp.ones((40, CW), jnp.float32)
    ones2 = jnp.ones((64, CW), jnp.float32)

    parts1 = _make_sc_agg(E1, N1P, 40, 5)(x, src1, dst1, z1)
    cnts1 = _make_sc_cnt(E1, N1P, 40, 8)(dst1.reshape(-1, 40), ones1, zc1)
    he = _dense(parts1, cnts1, x[:N1], W_l1.T, W_r1.T, b_l1[None, :],
                relu=True, BR=1000)
    parts2 = _make_sc_agg(E2, N2, 64, 8)(he, src2, dst2, z2)
    cnts2 = _make_sc_cnt(E2, N2, 64, 8)(dst2.reshape(-1, 64), ones2, zc2)
    h2 = _dense(parts2, cnts2, he[:N2], W_l2.T, W_r2.T, b_l2[None, :],
                relu=False, BR=1024)
    return (h2, h2, he)


# speedup vs baseline: 2.5226x; 1.0012x over previous
import functools

import jax
import jax.numpy as jnp
from jax import lax
from jax.experimental import pallas as pl
from jax.experimental.pallas import tpu as pltpu
from jax.experimental.pallas import tpu_sc as plsc

N0, N1, N2 = 50000, 10000, 4096
E1, E2 = 320000, 131072
D = 128
CW = 16
NC, NS = 2, 16


def _make_sc_agg(E, NP, C, S):
    EW = E // (NC * NS)
    n_chunks = EW // C
    assert n_chunks * C == EW
    n_rounds = n_chunks // S
    assert n_rounds * S == n_chunks and n_rounds >= 2
    RPT = NP // NS
    assert RPT * NS == NP and RPT % 8 == 0
    mesh = plsc.VectorSubcoreMesh(core_axis_name="c", subcore_axis_name="s",
                                  num_cores=NC, num_subcores=NS)

    @functools.partial(
        pl.kernel,
        out_type=jax.ShapeDtypeStruct((NC, NP, D), jnp.float32),
        mesh=mesh,
        scratch_types=(
            [pltpu.VMEM((S, C, D), jnp.float32)]
            + [pltpu.VMEM((C,), jnp.int32)] * S
            + [pltpu.VMEM((C,), jnp.int32)] * S
            + [pltpu.VMEM_SHARED((NP, D), jnp.float32)]
            + [pltpu.SemaphoreType.DMA] * (3 * S)
        ),
    )
    def agg_kernel(table, src, dst, zeros, out, bufs, *rest):
        idxs = rest[:S]
        idxd = rest[S:2 * S]
        acc_sh = rest[2 * S]
        isem = rest[2 * S + 1:2 * S + 1 + S]
        gsem = rest[2 * S + 1 + S:2 * S + 1 + 2 * S]
        ssem = rest[2 * S + 1 + 2 * S:]
        cid = lax.axis_index("c")
        sid = lax.axis_index("s")
        base = (cid * NS + sid) * EW
        pltpu.sync_copy(zeros.at[pl.ds(sid * RPT, RPT)],
                        acc_sh.at[pl.ds(sid * RPT, RPT)])
        plsc.subcore_barrier()

        def round_body(r, carry):
            for s in range(S):
                c = r * S + s

                @pl.when(r > 0)
                def _():
                    pltpu.make_async_copy(
                        bufs.at[s], acc_sh.at[idxd[s]], ssem[s]).wait()
                pltpu.async_copy(src.at[pl.ds(base + c * C, C)],
                                 idxs[s], isem[s])
                pltpu.async_copy(dst.at[pl.ds(base + c * C, C)],
                                 idxd[s], isem[s])
            for s in range(S):
                c = r * S + s
                pltpu.make_async_copy(src.at[pl.ds(base + c * C, C)],
                                      idxs[s], isem[s]).wait()
                pltpu.make_async_copy(dst.at[pl.ds(base + c * C, C)],
                                      idxd[s], isem[s]).wait()
                pltpu.async_copy(table.at[idxs[s]], bufs.at[s], gsem[s])
            for s in range(S):
                pltpu.make_async_copy(table.at[idxs[s]], bufs.at[s],
                                      gsem[s]).wait()
                pltpu.async_copy(bufs.at[s], acc_sh.at[idxd[s]], ssem[s],
                                 add=True)
            return carry

        lax.fori_loop(0, n_rounds, round_body, 0)
        for s in range(S):
            pltpu.make_async_copy(bufs.at[s], acc_sh.at[idxd[s]],
                                  ssem[s]).wait()
        plsc.subcore_barrier()
        pltpu.sync_copy(acc_sh.at[pl.ds(sid * RPT, RPT)],
                        out.at[cid, pl.ds(sid * RPT, RPT)])

    return agg_kernel


def _make_sc_cnt(E, NP, C, S):
    EW = E // (NC * NS)
    n_chunks = EW // C
    assert n_chunks * C == EW
    RPT = NP // NS
    assert RPT * NS == NP and RPT % 8 == 0
    mesh = plsc.VectorSubcoreMesh(core_axis_name="c", subcore_axis_name="s",
                                  num_cores=NC, num_subcores=NS)

    @functools.partial(
        pl.kernel,
        out_type=jax.ShapeDtypeStruct((NC, NP, CW), jnp.float32),
        mesh=mesh,
        scratch_types=[
            pltpu.VMEM((n_chunks, C), jnp.int32),
            pltpu.VMEM((C, CW), jnp.float32),
            pltpu.VMEM_SHARED((NP, CW), jnp.float32),
            pltpu.SemaphoreType.DMA,
        ],
        compiler_params=pltpu.CompilerParams(use_tc_tiling_on_sc=False),
    )
    def cnt_kernel(dstR, ones, zeros, out, idxd_v, ones_v, acc_sh, ssem):
        cid = lax.axis_index("c")
        sid = lax.axis_index("s")
        w = cid * NS + sid
        pltpu.sync_copy(zeros.at[pl.ds(sid * RPT, RPT)],
                        acc_sh.at[pl.ds(sid * RPT, RPT)])
        pltpu.sync_copy(dstR.at[pl.ds(w * n_chunks, n_chunks)], idxd_v)
        pltpu.sync_copy(ones, ones_v)
        plsc.subcore_barrier()

        def scatter(c):
            return pltpu.async_copy(ones_v, acc_sh.at[idxd_v.at[c]], ssem,
                                    add=True)

        for c in range(S):
            scatter(c)

        def body(c, carry):
            pltpu.make_async_copy(ones_v, acc_sh.at[idxd_v.at[c]],
                                  ssem).wait()
            scatter(c + S)
            return carry

        lax.fori_loop(0, n_chunks - S, body, 0)
        for c in range(S):
            pltpu.make_async_copy(ones_v, acc_sh.at[idxd_v.at[0]],
                                  ssem).wait()
        plsc.subcore_barrier()
        pltpu.sync_copy(acc_sh.at[pl.ds(sid * RPT, RPT)],
                        out.at[cid, pl.ds(sid * RPT, RPT)])

    return cnt_kernel


def _dense(parts, cnts, xdst, wlT, wrT, b, relu, BR, N):
    assert N % BR == 0

    def body(p_ref, c_ref, xd_ref, wl_ref, wr_ref, b_ref, o_ref):
        agg = p_ref[0] + p_ref[1]
        cnt = jnp.maximum(c_ref[0, :, 0:1] + c_ref[1, :, 0:1], 1.0)
        mean = agg / cnt
        h = jnp.dot(mean, wl_ref[...], preferred_element_type=jnp.float32)
        h = h + jnp.dot(xd_ref[...], wr_ref[...],
                        preferred_element_type=jnp.float32)
        h = h + b_ref[...]
        if relu:
            h = jnp.maximum(h, 0.0)
        o_ref[...] = h

    return pl.pallas_call(
        body,
        grid=(N // BR,),
        in_specs=[
            pl.BlockSpec((NC, BR, D), lambda i: (0, i, 0)),
            pl.BlockSpec((NC, BR, CW), lambda i: (0, i, 0)),
            pl.BlockSpec((BR, D), lambda i: (i, 0)),
            pl.BlockSpec((D, D), lambda i: (0, 0)),
            pl.BlockSpec((D, D), lambda i: (0, 0)),
            pl.BlockSpec((1, D), lambda i: (0, 0)),
        ],
        out_specs=pl.BlockSpec((BR, D), lambda i: (i, 0)),
        out_shape=jax.ShapeDtypeStruct((N, D), jnp.float32),
    )(parts, cnts, xdst, wlT, wrT, b)


def kernel(x, edge_index1, edge_index2, W_l1, b_l1, W_r1, W_l2, b_l2, W_r2):
    src1 = edge_index1[0].astype(jnp.int32)
    dst1 = edge_index1[1].astype(jnp.int32)
    src2 = edge_index2[0].astype(jnp.int32)
    dst2 = edge_index2[1].astype(jnp.int32)

    N1P = 10112
    z1 = jnp.zeros((N1P, D), jnp.float32)
    z2 = jnp.zeros((N2, D), jnp.float32)
    zc1 = jnp.zeros((N1P, CW), jnp.float32)
    zc2 = jnp.zeros((N2, CW), jnp.float32)
    ones1 = jnp.ones((40, CW), jnp.float32)
    ones2 = jnp.ones((64, CW), jnp.float32)

    parts1 = _make_sc_agg(E1, N1P, 40, 5)(x, src1, dst1, z1)
    cnts1 = _make_sc_cnt(E1, N1P, 40, 8)(dst1.reshape(-1, 40), ones1, zc1)
    he = _dense(parts1, cnts1, x, W_l1.T, W_r1.T, b_l1[None, :],
                relu=True, BR=1000, N=N1)
    parts2 = _make_sc_agg(E2, N2, 64, 8)(he, src2, dst2, z2)
    cnts2 = _make_sc_cnt(E2, N2, 64, 8)(dst2.reshape(-1, 64), ones2, zc2)
    h2 = _dense(parts2, cnts2, he, W_l2.T, W_r2.T, b_l2[None, :],
                relu=False, BR=1024, N=N2)
    return (h2, h2, he)


# speedup vs baseline: 3.2305x; 1.2806x over previous
import functools

import jax
import jax.numpy as jnp
from jax import lax
from jax.experimental import pallas as pl
from jax.experimental.pallas import tpu as pltpu
from jax.experimental.pallas import tpu_sc as plsc

N0, N1, N2 = 50000, 10000, 4096
E1, E2 = 320000, 131072
D = 128
CW = 16
NC, NS = 2, 16


def _make_sc_agg(E, NP, C, S):
    EW = E // (NC * NS)
    n_chunks = EW // C
    assert n_chunks * C == EW
    n_rounds = n_chunks // S
    assert n_rounds * S == n_chunks and n_rounds >= 2
    RPT = NP // NS
    assert RPT * NS == NP and RPT % 8 == 0
    mesh = plsc.VectorSubcoreMesh(core_axis_name="c", subcore_axis_name="s",
                                  num_cores=NC, num_subcores=NS)

    @functools.partial(
        pl.kernel,
        out_type=jax.ShapeDtypeStruct((NC, NP, D), jnp.float32),
        mesh=mesh,
        scratch_types=(
            [pltpu.VMEM((S, C, D), jnp.float32),
             pltpu.VMEM((EW,), jnp.int32),
             pltpu.VMEM((EW,), jnp.int32),
             pltpu.VMEM_SHARED((NP, D), jnp.float32)]
            + [pltpu.SemaphoreType.DMA] * (2 * S)
        ),
    )
    def agg_kernel(table, src, dst, zeros, out, bufs, idxs_v, idxd_v,
                   acc_sh, *sems):
        gsem = sems[:S]
        ssem = sems[S:]
        cid = lax.axis_index("c")
        sid = lax.axis_index("s")
        base = (cid * NS + sid) * EW
        pltpu.sync_copy(src.at[pl.ds(base, EW)], idxs_v)
        pltpu.sync_copy(dst.at[pl.ds(base, EW)], idxd_v)
        pltpu.sync_copy(zeros.at[pl.ds(sid * RPT, RPT)],
                        acc_sh.at[pl.ds(sid * RPT, RPT)])
        plsc.subcore_barrier()

        def gather(c, s):
            return pltpu.async_copy(table.at[idxs_v.at[pl.ds(c * C, C)]],
                                    bufs.at[s], gsem[s])

        def scatter(c, s):
            return pltpu.async_copy(bufs.at[s],
                                    acc_sh.at[idxd_v.at[pl.ds(c * C, C)]],
                                    ssem[s], add=True)

        for s in range(S):
            gather(s, s)

        def round_body(r, carry):
            for s in range(S):
                c = r * S + s
                pltpu.make_async_copy(table.at[idxs_v.at[pl.ds(c * C, C)]],
                                      bufs.at[s], gsem[s]).wait()
                scatter(c, s)
                pltpu.make_async_copy(bufs.at[s],
                                      acc_sh.at[idxd_v.at[pl.ds(c * C, C)]],
                                      ssem[s]).wait()

                @pl.when(r < n_rounds - 1)
                def _():
                    gather(c + S, s)
            return carry

        lax.fori_loop(0, n_rounds, round_body, 0)
        plsc.subcore_barrier()
        pltpu.sync_copy(acc_sh.at[pl.ds(sid * RPT, RPT)],
                        out.at[cid, pl.ds(sid * RPT, RPT)])

    return agg_kernel


def _make_sc_cnt(E, NP, C, S):
    EW = E // (NC * NS)
    n_chunks = EW // C
    assert n_chunks * C == EW
    RPT = NP // NS
    assert RPT * NS == NP and RPT % 8 == 0
    mesh = plsc.VectorSubcoreMesh(core_axis_name="c", subcore_axis_name="s",
                                  num_cores=NC, num_subcores=NS)

    @functools.partial(
        pl.kernel,
        out_type=jax.ShapeDtypeStruct((NC, NP, CW), jnp.float32),
        mesh=mesh,
        scratch_types=[
            pltpu.VMEM((n_chunks, C), jnp.int32),
            pltpu.VMEM((C, CW), jnp.float32),
            pltpu.VMEM_SHARED((NP, CW), jnp.float32),
            pltpu.SemaphoreType.DMA,
        ],
        compiler_params=pltpu.CompilerParams(use_tc_tiling_on_sc=False),
    )
    def cnt_kernel(dstR, ones, zeros, out, idxd_v, ones_v, acc_sh, ssem):
        cid = lax.axis_index("c")
        sid = lax.axis_index("s")
        w = cid * NS + sid
        pltpu.sync_copy(zeros.at[pl.ds(sid * RPT, RPT)],
                        acc_sh.at[pl.ds(sid * RPT, RPT)])
        pltpu.sync_copy(dstR.at[pl.ds(w * n_chunks, n_chunks)], idxd_v)
        pltpu.sync_copy(ones, ones_v)
        plsc.subcore_barrier()

        def scatter(c):
            return pltpu.async_copy(ones_v, acc_sh.at[idxd_v.at[c]], ssem,
                                    add=True)

        for c in range(S):
            scatter(c)

        def body(c, carry):
            pltpu.make_async_copy(ones_v, acc_sh.at[idxd_v.at[c]],
                                  ssem).wait()
            scatter(c + S)
            return carry

        lax.fori_loop(0, n_chunks - S, body, 0)
        for c in range(S):
            pltpu.make_async_copy(ones_v, acc_sh.at[idxd_v.at[0]],
                                  ssem).wait()
        plsc.subcore_barrier()
        pltpu.sync_copy(acc_sh.at[pl.ds(sid * RPT, RPT)],
                        out.at[cid, pl.ds(sid * RPT, RPT)])

    return cnt_kernel


def _dense(parts, cnts, xdst, wlT, wrT, b, relu, BR, N):
    assert N % BR == 0

    def body(p_ref, c_ref, xd_ref, wl_ref, wr_ref, b_ref, o_ref):
        agg = p_ref[0] + p_ref[1]
        cnt = jnp.maximum(c_ref[0, :, 0:1] + c_ref[1, :, 0:1], 1.0)
        mean = agg / cnt
        h = jnp.dot(mean, wl_ref[...], preferred_element_type=jnp.float32)
        h = h + jnp.dot(xd_ref[...], wr_ref[...],
                        preferred_element_type=jnp.float32)
        h = h + b_ref[...]
        if relu:
            h = jnp.maximum(h, 0.0)
        o_ref[...] = h

    return pl.pallas_call(
        body,
        grid=(N // BR,),
        in_specs=[
            pl.BlockSpec((NC, BR, D), lambda i: (0, i, 0)),
            pl.BlockSpec((NC, BR, CW), lambda i: (0, i, 0)),
            pl.BlockSpec((BR, D), lambda i: (i, 0)),
            pl.BlockSpec((D, D), lambda i: (0, 0)),
            pl.BlockSpec((D, D), lambda i: (0, 0)),
            pl.BlockSpec((1, D), lambda i: (0, 0)),
        ],
        out_specs=pl.BlockSpec((BR, D), lambda i: (i, 0)),
        out_shape=jax.ShapeDtypeStruct((N, D), jnp.float32),
    )(parts, cnts, xdst, wlT, wrT, b)


def kernel(x, edge_index1, edge_index2, W_l1, b_l1, W_r1, W_l2, b_l2, W_r2):
    src1 = edge_index1[0].astype(jnp.int32)
    dst1 = edge_index1[1].astype(jnp.int32)
    src2 = edge_index2[0].astype(jnp.int32)
    dst2 = edge_index2[1].astype(jnp.int32)

    N1P = 10112
    z1 = jnp.zeros((N1P, D), jnp.float32)
    z2 = jnp.zeros((N2, D), jnp.float32)
    zc1 = jnp.zeros((N1P, CW), jnp.float32)
    zc2 = jnp.zeros((N2, CW), jnp.float32)
    ones1 = jnp.ones((40, CW), jnp.float32)
    ones2 = jnp.ones((64, CW), jnp.float32)

    parts1 = _make_sc_agg(E1, N1P, 40, 5)(x, src1, dst1, z1)
    cnts1 = _make_sc_cnt(E1, N1P, 40, 8)(dst1.reshape(-1, 40), ones1, zc1)
    he = _dense(parts1, cnts1, x, W_l1.T, W_r1.T, b_l1[None, :],
                relu=True, BR=1000, N=N1)
    parts2 = _make_sc_agg(E2, N2, 64, 8)(he, src2, dst2, z2)
    cnts2 = _make_sc_cnt(E2, N2, 64, 8)(dst2.reshape(-1, 64), ones2, zc2)
    h2 = _dense(parts2, cnts2, he, W_l2.T, W_r2.T, b_l2[None, :],
                relu=False, BR=1024, N=N2)
    return (h2, h2, he)
